# baseline (speedup 1.0000x reference)
import functools

import jax
import jax.numpy as jnp
from jax import lax
from jax.experimental import pallas as pl
from jax.experimental.pallas import tpu as pltpu
from jax.experimental.pallas import tpu_sc as plsc

NC = 2
NS = 16
NW = NC * NS
_C = 125
_DW = 16
_BLK = 400
_G = 64
F32 = jnp.float32


def _deg_body(n, cpw, dst_hbm, zeros_hbm, ones_hbm, degp_hbm,
              didx, zbuf, obuf, deg_sh):
    c = lax.axis_index("c")
    s = lax.axis_index("s")
    wid = c * NS + s
    rpt = n // NS
    nch = rpt // _C
    pltpu.sync_copy(zeros_hbm, zbuf)
    pltpu.sync_copy(ones_hbm, obuf)
    pltpu.sync_copy(dst_hbm.at[wid], didx)
    for k in range(nch):
        pltpu.sync_copy(zbuf, deg_sh.at[pl.ds(s * rpt + k * _C, _C)])
    plsc.subcore_barrier()

    def step(j, carry):
        pltpu.sync_copy(obuf, deg_sh.at[didx.at[j]], add=True)
        return carry

    lax.fori_loop(0, cpw, step, 0)
    plsc.subcore_barrier()
    for k in range(nch):
        r0 = s * rpt + k * _C
        pltpu.sync_copy(deg_sh.at[pl.ds(r0, _C)], zbuf)
        pltpu.sync_copy(zbuf, degp_hbm.at[c, pl.ds(r0, _C)])


def _agg_body(n, h, cpw, y_hbm, src_hbm, dst_hbm, accp_hbm,
              sidx, didx, rows0, rows1, gsem0, gsem1, ssem, acc_sh):
    c = lax.axis_index("c")
    s = lax.axis_index("s")
    wid = c * NS + s
    rpt = n // NS
    nch = rpt // _C
    half = cpw // 2
    rows = (rows0, rows1)
    gsems = (gsem0, gsem1)

    pltpu.async_copy(src_hbm.at[wid, pl.ds(0, half)], sidx, gsem0)
    pltpu.async_copy(dst_hbm.at[wid, pl.ds(0, half)], didx, gsem1)

    def zrow(i, carry):
        for k8 in range(h // 16):
            rows0[i, pl.ds(k8 * 16, 16)] = jnp.zeros((16,), F32)
        return carry

    lax.fori_loop(0, _C, zrow, 0)
    for k in range(nch):
        pltpu.async_copy(rows0, acc_sh.at[pl.ds(s * rpt + k * _C, _C)], ssem)
    for k in range(nch):
        pltpu.make_async_copy(rows0, acc_sh.at[pl.ds(0, _C)], ssem).wait()
    pltpu.make_async_copy(src_hbm.at[wid, pl.ds(0, half)], sidx, gsem0).wait()
    pltpu.make_async_copy(dst_hbm.at[wid, pl.ds(0, half)], didx, gsem1).wait()
    plsc.subcore_barrier()

    pltpu.async_copy(y_hbm.at[sidx.at[0]], rows0, gsem0)

    def g_wait(b):
        pltpu.make_async_copy(y_hbm.at[sidx.at[0]], rows[b], gsems[b]).wait()

    def s_wait():
        pltpu.make_async_copy(rows[0], acc_sh.at[didx.at[0]], ssem).wait()

    g_wait(0)
    pltpu.async_copy(rows0, acc_sh.at[didx.at[0]], ssem, add=True)
    pltpu.async_copy(y_hbm.at[sidx.at[1]], rows1, gsem1)

    def pair(i, carry):
        for off in range(2):
            k = 2 * i + 1 + off
            b = (1 + off) % 2
            g_wait(b)
            s_wait()
            if off == 0:
                @pl.when(k == half - 1)
                def _():
                    pltpu.sync_copy(src_hbm.at[wid, pl.ds(half, half)], sidx)
            else:
                @pl.when(k == half)
                def _():
                    pltpu.sync_copy(dst_hbm.at[wid, pl.ds(half, half)], didx)
            kl = k - half * (k // half)
            pltpu.async_copy(rows[b], acc_sh.at[didx.at[kl]], ssem, add=True)
            k1 = k + 1
            kg = k1 - half * (k1 // half)
            pltpu.async_copy(y_hbm.at[sidx.at[kg]], rows[1 - b],
                             gsems[1 - b])
        return carry

    lax.fori_loop(0, (cpw - 2) // 2, pair, 0)
    g_wait((cpw - 1) % 2)
    s_wait()
    pltpu.async_copy(rows[(cpw - 1) % 2],
                     acc_sh.at[didx.at[half - 1]], ssem, add=True)
    s_wait()
    plsc.subcore_barrier()
    for k in range(nch):
        r0 = s * rpt + k * _C
        if k >= 2:
            pltpu.make_async_copy(
                rows[k % 2], accp_hbm.at[c, pl.ds(0, _C)], gsems[k % 2]).wait()
        pltpu.sync_copy(acc_sh.at[pl.ds(r0, _C)], rows[k % 2])
        pltpu.async_copy(rows[k % 2], accp_hbm.at[c, pl.ds(r0, _C)],
                         gsems[k % 2])
    pltpu.make_async_copy(
        rows[(nch - 2) % 2], accp_hbm.at[c, pl.ds(0, _C)],
        gsems[(nch - 2) % 2]).wait()
    pltpu.make_async_copy(
        rows[(nch - 1) % 2], accp_hbm.at[c, pl.ds(0, _C)],
        gsems[(nch - 1) % 2]).wait()


def _make_deg(n, cpw):
    mesh = plsc.VectorSubcoreMesh(core_axis_name="c", subcore_axis_name="s")
    return pl.kernel(
        functools.partial(_deg_body, n, cpw),
        out_type=jax.ShapeDtypeStruct((NC, n, _DW), F32),
        mesh=mesh,
        scratch_types=[
            pltpu.VMEM((cpw, _C), jnp.int32),
            pltpu.VMEM((_C, _DW), F32),
            pltpu.VMEM((_C, _DW), F32),
            pltpu.VMEM_SHARED((n, _DW), F32),
        ],
        compiler_params=pltpu.CompilerParams(use_tc_tiling_on_sc=False),
    )


def _make_agg(n, h, cpw):
    mesh = plsc.VectorSubcoreMesh(core_axis_name="c", subcore_axis_name="s")
    return pl.kernel(
        functools.partial(_agg_body, n, h, cpw),
        out_type=jax.ShapeDtypeStruct((NC, n, h), F32),
        mesh=mesh,
        scratch_types=[
            pltpu.VMEM((cpw // 2, _C), jnp.int32),
            pltpu.VMEM((cpw // 2, _C), jnp.int32),
            pltpu.VMEM((_C, h), F32),
            pltpu.VMEM((_C, h), F32),
            pltpu.SemaphoreType.DMA,
            pltpu.SemaphoreType.DMA,
            pltpu.SemaphoreType.DMA,
            pltpu.VMEM_SHARED((n, h), F32),
        ],
        compiler_params=pltpu.CompilerParams(use_tc_tiling_on_sc=False),
    )


def _tc1a_body(x_ref, w_ref, xw_ref):
    xw_ref[...] = jnp.dot(x_ref[...], w_ref[...], preferred_element_type=F32)


def _tc1b_body(xw_ref, degp_ref, y_ref, dinv_ref):
    deg = degp_ref[0, :, 0:1] + degp_ref[1, :, 0:1] + 1.0
    di = lax.rsqrt(deg)
    dinv_ref[...] = di
    y_ref[...] = xw_ref[...] * di


def _tc_mid_body(accp_ref, y_ref, dinv_ref, b_ref, w_ref, out_ref):
    di = dinv_ref[...]
    hcur = jnp.maximum(
        di * (accp_ref[0] + accp_ref[1] + y_ref[...]) + b_ref[...], 0.0)
    out_ref[...] = jnp.dot(hcur, w_ref[...], preferred_element_type=F32) * di


def _tc_fin_body(nblk, accp_ref, y_ref, dinv_ref, b_ref, batch_ref,
                 wfc_ref, bfc_ref, out_ref, sums, counts):
    i = pl.program_id(0)

    @pl.when(i == 0)
    def _():
        sums[...] = jnp.zeros_like(sums)
        counts[...] = jnp.zeros_like(counts)

    di = dinv_ref[...]
    hcur = jnp.maximum(
        di * (accp_ref[0] + accp_ref[1] + y_ref[...]) + b_ref[...], 0.0)
    gid = lax.broadcasted_iota(jnp.int32, (hcur.shape[0], _G), 1).astype(F32)
    sel = (batch_ref[...] == gid).astype(F32)
    sums[...] += lax.dot_general(sel, hcur, (((0,), (0,)), ((), ())),
                                 preferred_element_type=F32)
    counts[...] += jnp.broadcast_to(jnp.sum(sel, axis=0)[:, None],
                                    counts.shape)

    @pl.when(i == nblk - 1)
    def _():
        pooled = sums[...] / jnp.maximum(counts[...], 1.0)
        out_ref[...] = jnp.dot(pooled, wfc_ref[...],
                               preferred_element_type=F32) + bfc_ref[...]


def _tc1a(x, w, n, d, h, nblk):
    return pl.pallas_call(
        _tc1a_body,
        grid=(nblk,),
        in_specs=[
            pl.BlockSpec((_BLK, d), lambda i: (i, 0)),
            pl.BlockSpec((d, h), lambda i: (0, 0)),
        ],
        out_specs=pl.BlockSpec((_BLK, h), lambda i: (i, 0)),
        out_shape=jax.ShapeDtypeStruct((n, h), F32),
    )(x, w)


def _tc1b(xw, degp, n, h, nblk):
    return pl.pallas_call(
        _tc1b_body,
        grid=(nblk,),
        in_specs=[
            pl.BlockSpec((_BLK, h), lambda i: (i, 0)),
            pl.BlockSpec((NC, _BLK, _DW), lambda i: (0, i, 0)),
        ],
        out_specs=[
            pl.BlockSpec((_BLK, h), lambda i: (i, 0)),
            pl.BlockSpec((_BLK, 1), lambda i: (i, 0)),
        ],
        out_shape=[
            jax.ShapeDtypeStruct((n, h), F32),
            jax.ShapeDtypeStruct((n, 1), F32),
        ],
    )(xw, degp)


def _tc_mid(accp, y, dinv, b, w, n, h, nblk):
    return pl.pallas_call(
        _tc_mid_body,
        grid=(nblk,),
        in_specs=[
            pl.BlockSpec((NC, _BLK, h), lambda i: (0, i, 0)),
            pl.BlockSpec((_BLK, h), lambda i: (i, 0)),
            pl.BlockSpec((_BLK, 1), lambda i: (i, 0)),
            pl.BlockSpec((1, h), lambda i: (0, 0)),
            pl.BlockSpec((h, h), lambda i: (0, 0)),
        ],
        out_specs=pl.BlockSpec((_BLK, h), lambda i: (i, 0)),
        out_shape=jax.ShapeDtypeStruct((n, h), F32),
    )(accp, y, dinv, b, w)


def _tc_fin(accp, y, dinv, b, batchf, wfc, bfc, n, h, nout, nblk):
    return pl.pallas_call(
        functools.partial(_tc_fin_body, nblk),
        grid=(nblk,),
        in_specs=[
            pl.BlockSpec((NC, _BLK, h), lambda i: (0, i, 0)),
            pl.BlockSpec((_BLK, h), lambda i: (i, 0)),
            pl.BlockSpec((_BLK, 1), lambda i: (i, 0)),
            pl.BlockSpec((1, h), lambda i: (0, 0)),
            pl.BlockSpec((_BLK, 1), lambda i: (i, 0)),
            pl.BlockSpec((h, nout), lambda i: (0, 0)),
            pl.BlockSpec((1, nout), lambda i: (0, 0)),
        ],
        out_specs=pl.BlockSpec((_G, nout), lambda i: (0, 0)),
        out_shape=jax.ShapeDtypeStruct((_G, nout), F32),
        scratch_shapes=[
            pltpu.VMEM((_G, h), F32),
            pltpu.VMEM((_G, h), F32),
        ],
    )(accp, y, dinv, b, batchf, wfc, bfc)


def kernel(x, edge_index, batch, W1, b1, W2, b2, W3, b3, Wfc, bfc):
    n, d = x.shape
    h = W1.shape[1]
    e = edge_index.shape[1]
    nout = Wfc.shape[1]
    assert e % (NW * _C) == 0 and n % (NS * _C) == 0 and n % _BLK == 0
    cpw = e // (NW * _C)
    assert cpw % 4 == 0 and cpw >= 8
    nblk = n // _BLK

    src3 = edge_index[0].reshape(NW, cpw, _C)
    dst3 = edge_index[1].reshape(NW, cpw, _C)
    zeros_t = jnp.zeros((_C, _DW), F32)
    ones_t = jnp.ones((_C, _DW), F32)

    deg_call = _make_deg(n, cpw)
    agg_call = _make_agg(n, h, cpw)

    degp = deg_call(dst3, zeros_t, ones_t)
    xw1 = _tc1a(x.astype(F32), W1, n, d, h, nblk)
    y1, dinv = _tc1b(xw1, degp, n, h, nblk)
    p1 = agg_call(y1, src3, dst3)
    y2 = _tc_mid(p1, y1, dinv, b1.reshape(1, h), W2, n, h, nblk)
    p2 = agg_call(y2, src3, dst3)
    y3 = _tc_mid(p2, y2, dinv, b2.reshape(1, h), W3, n, h, nblk)
    p3 = agg_call(y3, src3, dst3)
    batchf = batch.astype(F32).reshape(n, 1)
    return _tc_fin(p3, y3, dinv, b3.reshape(1, h), batchf, Wfc,
                   bfc.reshape(1, nout), n, h, nout, nblk)


# speedup vs baseline: 1.0016x; 1.0016x over previous
import functools

import jax
import jax.numpy as jnp
from jax import lax
from jax.experimental import pallas as pl
from jax.experimental.pallas import tpu as pltpu
from jax.experimental.pallas import tpu_sc as plsc

NC = 2
NS = 16
NW = NC * NS
_C = 128
_DC = 125
_DW = 16
_BLK = 400
_G = 64
F32 = jnp.float32


def _deg_body(n, npad, cpw, dst_hbm, zeros_hbm, ones_hbm, degp_hbm,
              didx, zbuf, obuf, deg_sh):
    c = lax.axis_index("c")
    s = lax.axis_index("s")
    wid = c * NS + s
    pltpu.sync_copy(zeros_hbm, zbuf)
    pltpu.sync_copy(ones_hbm, obuf)
    pltpu.sync_copy(dst_hbm.at[wid], didx)
    for k in range(npad // NS // _C):
        pltpu.sync_copy(zbuf, deg_sh.at[pl.ds(s * (npad // NS) + k * _C, _C)])
    plsc.subcore_barrier()

    def step(j, carry):
        pltpu.sync_copy(obuf, deg_sh.at[didx.at[j]], add=True)
        return carry

    lax.fori_loop(0, cpw, step, 0)
    plsc.subcore_barrier()
    for k in range(n // NS // _DC):
        r0 = s * (n // NS) + k * _DC
        pltpu.sync_copy(deg_sh.at[pl.ds(r0, _DC)], zbuf.at[pl.ds(0, _DC)])
        pltpu.sync_copy(zbuf.at[pl.ds(0, _DC)], degp_hbm.at[c, pl.ds(r0, _DC)])


def _agg_body(n, npad, h, cpw, y_hbm, src_hbm, dst_hbm, accp_hbm,
              sidx, didx, rows0, rows1, gsem0, gsem1, ssem, acc_sh):
    c = lax.axis_index("c")
    s = lax.axis_index("s")
    wid = c * NS + s
    half = cpw // 2
    rows = (rows0, rows1)
    gsems = (gsem0, gsem1)

    pltpu.async_copy(src_hbm.at[wid, pl.ds(0, half)], sidx, gsem0)
    pltpu.async_copy(dst_hbm.at[wid, pl.ds(0, half)], didx, gsem1)

    def zrow(i, carry):
        for k8 in range(h // 16):
            rows0[i, pl.ds(k8 * 16, 16)] = jnp.zeros((16,), F32)
        return carry

    lax.fori_loop(0, _C, zrow, 0)
    zch = npad // NS // _C
    for k in range(zch):
        pltpu.async_copy(
            rows0, acc_sh.at[pl.ds(s * (npad // NS) + k * _C, _C)], ssem)
    for k in range(zch):
        pltpu.make_async_copy(rows0, acc_sh.at[pl.ds(0, _C)], ssem).wait()
    pltpu.make_async_copy(src_hbm.at[wid, pl.ds(0, half)], sidx, gsem0).wait()
    pltpu.make_async_copy(dst_hbm.at[wid, pl.ds(0, half)], didx, gsem1).wait()
    plsc.subcore_barrier()

    pltpu.async_copy(y_hbm.at[sidx.at[0]], rows0, gsem0)

    def g_wait(b):
        pltpu.make_async_copy(y_hbm.at[sidx.at[0]], rows[b], gsems[b]).wait()

    def s_wait():
        pltpu.make_async_copy(rows[0], acc_sh.at[didx.at[0]], ssem).wait()

    g_wait(0)
    pltpu.async_copy(rows0, acc_sh.at[didx.at[0]], ssem, add=True)
    pltpu.async_copy(y_hbm.at[sidx.at[1]], rows1, gsem1)

    def pair(i, carry):
        for off in range(2):
            k = 2 * i + 1 + off
            b = (1 + off) % 2
            g_wait(b)
            s_wait()
            if off == 0:
                @pl.when(k == half - 1)
                def _():
                    pltpu.sync_copy(src_hbm.at[wid, pl.ds(half, half)], sidx)
            else:
                @pl.when(k == half)
                def _():
                    pltpu.sync_copy(dst_hbm.at[wid, pl.ds(half, half)], didx)
            kl = k - half * (k // half)
            pltpu.async_copy(rows[b], acc_sh.at[didx.at[kl]], ssem, add=True)
            k1 = k + 1
            kg = k1 - half * (k1 // half)
            pltpu.async_copy(y_hbm.at[sidx.at[kg]], rows[1 - b],
                             gsems[1 - b])
        return carry

    lax.fori_loop(0, (cpw - 2) // 2, pair, 0)
    g_wait((cpw - 1) % 2)
    s_wait()
    pltpu.async_copy(rows[(cpw - 1) % 2],
                     acc_sh.at[didx.at[half - 1]], ssem, add=True)
    s_wait()
    plsc.subcore_barrier()
    nch = n // NS // _DC
    for k in range(nch):
        r0 = s * (n // NS) + k * _DC
        if k >= 2:
            pltpu.make_async_copy(
                rows[k % 2].at[pl.ds(0, _DC)], accp_hbm.at[c, pl.ds(0, _DC)],
                gsems[k % 2]).wait()
        pltpu.sync_copy(acc_sh.at[pl.ds(r0, _DC)], rows[k % 2].at[pl.ds(0, _DC)])
        pltpu.async_copy(rows[k % 2].at[pl.ds(0, _DC)],
                         accp_hbm.at[c, pl.ds(r0, _DC)], gsems[k % 2])
    pltpu.make_async_copy(
        rows[(nch - 2) % 2].at[pl.ds(0, _DC)], accp_hbm.at[c, pl.ds(0, _DC)],
        gsems[(nch - 2) % 2]).wait()
    pltpu.make_async_copy(
        rows[(nch - 1) % 2].at[pl.ds(0, _DC)], accp_hbm.at[c, pl.ds(0, _DC)],
        gsems[(nch - 1) % 2]).wait()


def _make_deg(n, npad, cpw):
    mesh = plsc.VectorSubcoreMesh(core_axis_name="c", subcore_axis_name="s")
    return pl.kernel(
        functools.partial(_deg_body, n, npad, cpw),
        out_type=jax.ShapeDtypeStruct((NC, n, _DW), F32),
        mesh=mesh,
        scratch_types=[
            pltpu.VMEM((cpw, _C), jnp.int32),
            pltpu.VMEM((_C, _DW), F32),
            pltpu.VMEM((_C, _DW), F32),
            pltpu.VMEM_SHARED((npad, _DW), F32),
        ],
        compiler_params=pltpu.CompilerParams(use_tc_tiling_on_sc=False),
    )


def _make_agg(n, npad, h, cpw):
    mesh = plsc.VectorSubcoreMesh(core_axis_name="c", subcore_axis_name="s")
    return pl.kernel(
        functools.partial(_agg_body, n, npad, h, cpw),
        out_type=jax.ShapeDtypeStruct((NC, n, h), F32),
        mesh=mesh,
        scratch_types=[
            pltpu.VMEM((cpw // 2, _C), jnp.int32),
            pltpu.VMEM((cpw // 2, _C), jnp.int32),
            pltpu.VMEM((_C, h), F32),
            pltpu.VMEM((_C, h), F32),
            pltpu.SemaphoreType.DMA,
            pltpu.SemaphoreType.DMA,
            pltpu.SemaphoreType.DMA,
            pltpu.VMEM_SHARED((npad, h), F32),
        ],
        compiler_params=pltpu.CompilerParams(use_tc_tiling_on_sc=False),
    )


def _tc1a_body(x_ref, w_ref, xw_ref):
    xw_ref[...] = jnp.dot(x_ref[...], w_ref[...], preferred_element_type=F32)


def _tc1b_body(xw_ref, degp_ref, y_ref, dinv_ref):
    deg = degp_ref[0, :, 0:1] + degp_ref[1, :, 0:1] + 1.0
    di = lax.rsqrt(deg)
    dinv_ref[...] = di
    y_ref[...] = xw_ref[...] * di


def _tc_mid_body(accp_ref, y_ref, dinv_ref, b_ref, w_ref, out_ref):
    di = dinv_ref[...]
    hcur = jnp.maximum(
        di * (accp_ref[0] + accp_ref[1] + y_ref[...]) + b_ref[...], 0.0)
    out_ref[...] = jnp.dot(hcur, w_ref[...], preferred_element_type=F32) * di


def _tc_fin_body(nblk, accp_ref, y_ref, dinv_ref, b_ref, batch_ref,
                 wfc_ref, bfc_ref, out_ref, sums, counts):
    i = pl.program_id(0)

    @pl.when(i == 0)
    def _():
        sums[...] = jnp.zeros_like(sums)
        counts[...] = jnp.zeros_like(counts)

    di = dinv_ref[...]
    hcur = jnp.maximum(
        di * (accp_ref[0] + accp_ref[1] + y_ref[...]) + b_ref[...], 0.0)
    gid = lax.broadcasted_iota(jnp.int32, (_G, hcur.shape[0]), 0).astype(F32)
    sel_t = (batch_ref[0] == gid).astype(F32)
    sums[...] += lax.dot_general(sel_t, hcur, (((1,), (0,)), ((), ())),
                                 preferred_element_type=F32)
    counts[...] += jnp.broadcast_to(jnp.sum(sel_t, axis=1)[:, None],
                                    counts.shape)

    @pl.when(i == nblk - 1)
    def _():
        pooled = sums[...] / jnp.maximum(counts[...], 1.0)
        out_ref[...] = jnp.dot(pooled, wfc_ref[...],
                               preferred_element_type=F32) + bfc_ref[...]


def _tc1a(x, w, n, d, h, nblk):
    return pl.pallas_call(
        _tc1a_body,
        grid=(nblk,),
        in_specs=[
            pl.BlockSpec((_BLK, d), lambda i: (i, 0)),
            pl.BlockSpec((d, h), lambda i: (0, 0)),
        ],
        out_specs=pl.BlockSpec((_BLK, h), lambda i: (i, 0)),
        out_shape=jax.ShapeDtypeStruct((n, h), F32),
    )(x, w)


def _tc1b(xw, degp, n, h, nblk):
    return pl.pallas_call(
        _tc1b_body,
        grid=(nblk,),
        in_specs=[
            pl.BlockSpec((_BLK, h), lambda i: (i, 0)),
            pl.BlockSpec((NC, _BLK, _DW), lambda i: (0, i, 0)),
        ],
        out_specs=[
            pl.BlockSpec((_BLK, h), lambda i: (i, 0)),
            pl.BlockSpec((_BLK, 1), lambda i: (i, 0)),
        ],
        out_shape=[
            jax.ShapeDtypeStruct((n, h), F32),
            jax.ShapeDtypeStruct((n, 1), F32),
        ],
    )(xw, degp)


def _tc_mid(accp, y, dinv, b, w, n, h, nblk):
    return pl.pallas_call(
        _tc_mid_body,
        grid=(nblk,),
        in_specs=[
            pl.BlockSpec((NC, _BLK, h), lambda i: (0, i, 0)),
            pl.BlockSpec((_BLK, h), lambda i: (i, 0)),
            pl.BlockSpec((_BLK, 1), lambda i: (i, 0)),
            pl.BlockSpec((1, h), lambda i: (0, 0)),
            pl.BlockSpec((h, h), lambda i: (0, 0)),
        ],
        out_specs=pl.BlockSpec((_BLK, h), lambda i: (i, 0)),
        out_shape=jax.ShapeDtypeStruct((n, h), F32),
    )(accp, y, dinv, b, w)


def _tc_fin(accp, y, dinv, b, batchf, wfc, bfc, n, h, nout, nblk):
    return pl.pallas_call(
        functools.partial(_tc_fin_body, nblk),
        grid=(nblk,),
        in_specs=[
            pl.BlockSpec((NC, _BLK, h), lambda i: (0, i, 0)),
            pl.BlockSpec((_BLK, h), lambda i: (i, 0)),
            pl.BlockSpec((_BLK, 1), lambda i: (i, 0)),
            pl.BlockSpec((1, h), lambda i: (0, 0)),
            pl.BlockSpec((1, 1, _BLK), lambda i: (i, 0, 0)),
            pl.BlockSpec((h, nout), lambda i: (0, 0)),
            pl.BlockSpec((1, nout), lambda i: (0, 0)),
        ],
        out_specs=pl.BlockSpec((_G, nout), lambda i: (0, 0)),
        out_shape=jax.ShapeDtypeStruct((_G, nout), F32),
        scratch_shapes=[
            pltpu.VMEM((_G, h), F32),
            pltpu.VMEM((_G, h), F32),
        ],
    )(accp, y, dinv, b, batchf, wfc, bfc)


def kernel(x, edge_index, batch, W1, b1, W2, b2, W3, b3, Wfc, bfc):
    n, d = x.shape
    h = W1.shape[1]
    e = edge_index.shape[1]
    nout = Wfc.shape[1]
    assert n % _BLK == 0 and (n // NS) % _DC == 0
    nblk = n // _BLK
    blk = NW * _C
    cpw = ((e + blk - 1) // blk + 3) // 4 * 4
    ep = cpw * blk
    pad_e = ep - e
    assert cpw >= 8
    npad = ((n + NS * _C - 1) // (NS * _C)) * (NS * _C)
    assert npad > n

    src_f = edge_index[0]
    dst_f = edge_index[1]
    if pad_e:
        pad_src = (jnp.arange(pad_e, dtype=jnp.int32) * 977) % n
        src_f = jnp.concatenate([src_f, pad_src])
        dst_f = jnp.concatenate(
            [dst_f, jnp.full((pad_e,), n, dtype=jnp.int32)])
    src3 = src_f.reshape(NW, cpw, _C)
    dst3 = dst_f.reshape(NW, cpw, _C)
    zeros_t = jnp.zeros((_C, _DW), F32)
    ones_t = jnp.ones((_C, _DW), F32)

    deg_call = _make_deg(n, npad, cpw)
    agg_call = _make_agg(n, npad, h, cpw)

    degp = deg_call(dst3, zeros_t, ones_t)
    xw1 = _tc1a(x.astype(F32), W1, n, d, h, nblk)
    y1, dinv = _tc1b(xw1, degp, n, h, nblk)
    p1 = agg_call(y1, src3, dst3)
    y2 = _tc_mid(p1, y1, dinv, b1.reshape(1, h), W2, n, h, nblk)
    p2 = agg_call(y2, src3, dst3)
    y3 = _tc_mid(p2, y2, dinv, b2.reshape(1, h), W3, n, h, nblk)
    p3 = agg_call(y3, src3, dst3)
    batchf = batch.astype(F32).reshape(nblk, 1, _BLK)
    return _tc_fin(p3, y3, dinv, b3.reshape(1, h), batchf, Wfc,
                   bfc.reshape(1, nout), n, h, nout, nblk)


# speedup vs baseline: 1.0805x; 1.0787x over previous
import functools

import jax
import jax.numpy as jnp
from jax import lax
from jax.experimental import pallas as pl
from jax.experimental.pallas import tpu as pltpu
from jax.experimental.pallas import tpu_sc as plsc

NC = 2
NS = 16
NW = NC * NS
_C = 128
_DC = 125
_DW = 16
_BLK = 2000
_G = 64
F32 = jnp.float32


def _deg_body(n, npad, cpw, dst_hbm, zeros_hbm, ones_hbm, degp_hbm,
              didx, zbuf, obuf, deg_sh):
    c = lax.axis_index("c")
    s = lax.axis_index("s")
    wid = c * NS + s
    pltpu.sync_copy(zeros_hbm, zbuf)
    pltpu.sync_copy(ones_hbm, obuf)
    pltpu.sync_copy(dst_hbm.at[wid], didx)
    for k in range(npad // NS // _C):
        pltpu.sync_copy(zbuf, deg_sh.at[pl.ds(s * (npad // NS) + k * _C, _C)])
    plsc.subcore_barrier()

    def step(j, carry):
        pltpu.sync_copy(obuf, deg_sh.at[didx.at[j]], add=True)
        return carry

    lax.fori_loop(0, cpw, step, 0)
    plsc.subcore_barrier()
    for k in range(n // NS // _DC):
        r0 = s * (n // NS) + k * _DC
        pltpu.sync_copy(deg_sh.at[pl.ds(r0, _DC)], zbuf.at[pl.ds(0, _DC)])
        pltpu.sync_copy(zbuf.at[pl.ds(0, _DC)], degp_hbm.at[c, pl.ds(r0, _DC)])


def _agg_body(n, npad, h, cpw, y_hbm, src_hbm, dst_hbm, accp_hbm,
              sidx, didx, rows0, rows1, gsem0, gsem1, ssem, acc_sh):
    c = lax.axis_index("c")
    s = lax.axis_index("s")
    wid = c * NS + s
    half = cpw // 2
    rows = (rows0, rows1)
    gsems = (gsem0, gsem1)

    pltpu.async_copy(src_hbm.at[wid, pl.ds(0, half)], sidx, gsem0)
    pltpu.async_copy(dst_hbm.at[wid, pl.ds(0, half)], didx, gsem1)

    def zrow(i, carry):
        for k8 in range(h // 16):
            rows0[i, pl.ds(k8 * 16, 16)] = jnp.zeros((16,), F32)
        return carry

    lax.fori_loop(0, _C, zrow, 0)
    zch = npad // NS // _C
    for k in range(zch):
        pltpu.async_copy(
            rows0, acc_sh.at[pl.ds(s * (npad // NS) + k * _C, _C)], ssem)
    for k in range(zch):
        pltpu.make_async_copy(rows0, acc_sh.at[pl.ds(0, _C)], ssem).wait()
    pltpu.make_async_copy(src_hbm.at[wid, pl.ds(0, half)], sidx, gsem0).wait()
    pltpu.make_async_copy(dst_hbm.at[wid, pl.ds(0, half)], didx, gsem1).wait()
    plsc.subcore_barrier()

    pltpu.async_copy(y_hbm.at[sidx.at[0]], rows0, gsem0)

    def g_wait(b):
        pltpu.make_async_copy(y_hbm.at[sidx.at[0]], rows[b], gsems[b]).wait()

    def s_wait():
        pltpu.make_async_copy(rows[0], acc_sh.at[didx.at[0]], ssem).wait()

    g_wait(0)
    pltpu.async_copy(rows0, acc_sh.at[didx.at[0]], ssem, add=True)
    pltpu.async_copy(y_hbm.at[sidx.at[1]], rows1, gsem1)

    def pair(i, carry):
        for off in range(2):
            k = 2 * i + 1 + off
            b = (1 + off) % 2
            g_wait(b)
            s_wait()
            if off == 0:
                @pl.when(k == half - 1)
                def _():
                    pltpu.sync_copy(src_hbm.at[wid, pl.ds(half, half)], sidx)
            else:
                @pl.when(k == half)
                def _():
                    pltpu.sync_copy(dst_hbm.at[wid, pl.ds(half, half)], didx)
            kl = k - half * (k // half)
            pltpu.async_copy(rows[b], acc_sh.at[didx.at[kl]], ssem, add=True)
            k1 = k + 1
            kg = k1 - half * (k1 // half)
            pltpu.async_copy(y_hbm.at[sidx.at[kg]], rows[1 - b],
                             gsems[1 - b])
        return carry

    lax.fori_loop(0, (cpw - 2) // 2, pair, 0)
    g_wait((cpw - 1) % 2)
    s_wait()
    pltpu.async_copy(rows[(cpw - 1) % 2],
                     acc_sh.at[didx.at[half - 1]], ssem, add=True)
    s_wait()
    plsc.subcore_barrier()
    nch = n // NS // _DC
    for k in range(nch):
        r0 = s * (n // NS) + k * _DC
        if k >= 2:
            pltpu.make_async_copy(
                rows[k % 2].at[pl.ds(0, _DC)], accp_hbm.at[c, pl.ds(0, _DC)],
                gsems[k % 2]).wait()
        pltpu.sync_copy(acc_sh.at[pl.ds(r0, _DC)], rows[k % 2].at[pl.ds(0, _DC)])
        pltpu.async_copy(rows[k % 2].at[pl.ds(0, _DC)],
                         accp_hbm.at[c, pl.ds(r0, _DC)], gsems[k % 2])
    pltpu.make_async_copy(
        rows[(nch - 2) % 2].at[pl.ds(0, _DC)], accp_hbm.at[c, pl.ds(0, _DC)],
        gsems[(nch - 2) % 2]).wait()
    pltpu.make_async_copy(
        rows[(nch - 1) % 2].at[pl.ds(0, _DC)], accp_hbm.at[c, pl.ds(0, _DC)],
        gsems[(nch - 1) % 2]).wait()


def _make_deg(n, npad, cpw):
    mesh = plsc.VectorSubcoreMesh(core_axis_name="c", subcore_axis_name="s")
    return pl.kernel(
        functools.partial(_deg_body, n, npad, cpw),
        out_type=jax.ShapeDtypeStruct((NC, n, _DW), F32),
        mesh=mesh,
        scratch_types=[
            pltpu.VMEM((cpw, _C), jnp.int32),
            pltpu.VMEM((_C, _DW), F32),
            pltpu.VMEM((_C, _DW), F32),
            pltpu.VMEM_SHARED((npad, _DW), F32),
        ],
        compiler_params=pltpu.CompilerParams(use_tc_tiling_on_sc=False),
    )


def _make_agg(n, npad, h, cpw):
    mesh = plsc.VectorSubcoreMesh(core_axis_name="c", subcore_axis_name="s")
    return pl.kernel(
        functools.partial(_agg_body, n, npad, h, cpw),
        out_type=jax.ShapeDtypeStruct((NC, n, h), F32),
        mesh=mesh,
        scratch_types=[
            pltpu.VMEM((cpw // 2, _C), jnp.int32),
            pltpu.VMEM((cpw // 2, _C), jnp.int32),
            pltpu.VMEM((_C, h), F32),
            pltpu.VMEM((_C, h), F32),
            pltpu.SemaphoreType.DMA,
            pltpu.SemaphoreType.DMA,
            pltpu.SemaphoreType.DMA,
            pltpu.VMEM_SHARED((npad, h), F32),
        ],
        compiler_params=pltpu.CompilerParams(use_tc_tiling_on_sc=False),
    )


def _tc1a_body(x_ref, w_ref, xw_ref):
    xw_ref[...] = jnp.dot(x_ref[...], w_ref[...], preferred_element_type=F32)


def _tc1b_body(xw_ref, degp_ref, y_ref, dinv_ref):
    deg = degp_ref[0, :, 0:1] + degp_ref[1, :, 0:1] + 1.0
    di = lax.rsqrt(deg)
    dinv_ref[...] = di
    y_ref[...] = xw_ref[...] * di


def _tc_mid_body(accp_ref, y_ref, dinv_ref, b_ref, w_ref, out_ref):
    di = dinv_ref[...]
    hcur = jnp.maximum(
        di * (accp_ref[0] + accp_ref[1] + y_ref[...]) + b_ref[...], 0.0)
    out_ref[...] = jnp.dot(hcur, w_ref[...], preferred_element_type=F32) * di


def _tc_fin_body(nblk, accp_ref, y_ref, dinv_ref, b_ref, batch_ref,
                 wfc_ref, bfc_ref, out_ref, sums, counts):
    i = pl.program_id(0)

    @pl.when(i == 0)
    def _():
        sums[...] = jnp.zeros_like(sums)
        counts[...] = jnp.zeros_like(counts)

    di = dinv_ref[...]
    hcur = jnp.maximum(
        di * (accp_ref[0] + accp_ref[1] + y_ref[...]) + b_ref[...], 0.0)
    gid = lax.broadcasted_iota(jnp.int32, (_G, hcur.shape[0]), 0).astype(F32)
    sel_t = (batch_ref[0] == gid).astype(F32)
    sums[...] += lax.dot_general(sel_t, hcur, (((1,), (0,)), ((), ())),
                                 preferred_element_type=F32)
    counts[...] += jnp.broadcast_to(jnp.sum(sel_t, axis=1)[:, None],
                                    counts.shape)

    @pl.when(i == nblk - 1)
    def _():
        pooled = sums[...] / jnp.maximum(counts[...], 1.0)
        out_ref[...] = jnp.dot(pooled, wfc_ref[...],
                               preferred_element_type=F32) + bfc_ref[...]


def _tc1a(x, w, n, d, h, nblk):
    return pl.pallas_call(
        _tc1a_body,
        grid=(nblk,),
        in_specs=[
            pl.BlockSpec((_BLK, d), lambda i: (i, 0)),
            pl.BlockSpec((d, h), lambda i: (0, 0)),
        ],
        out_specs=pl.BlockSpec((_BLK, h), lambda i: (i, 0)),
        out_shape=jax.ShapeDtypeStruct((n, h), F32),
    )(x, w)


def _tc1b(xw, degp, n, h, nblk):
    return pl.pallas_call(
        _tc1b_body,
        grid=(nblk,),
        in_specs=[
            pl.BlockSpec((_BLK, h), lambda i: (i, 0)),
            pl.BlockSpec((NC, _BLK, _DW), lambda i: (0, i, 0)),
        ],
        out_specs=[
            pl.BlockSpec((_BLK, h), lambda i: (i, 0)),
            pl.BlockSpec((_BLK, 1), lambda i: (i, 0)),
        ],
        out_shape=[
            jax.ShapeDtypeStruct((n, h), F32),
            jax.ShapeDtypeStruct((n, 1), F32),
        ],
    )(xw, degp)


def _tc_mid(accp, y, dinv, b, w, n, h, nblk):
    return pl.pallas_call(
        _tc_mid_body,
        grid=(nblk,),
        in_specs=[
            pl.BlockSpec((NC, _BLK, h), lambda i: (0, i, 0)),
            pl.BlockSpec((_BLK, h), lambda i: (i, 0)),
            pl.BlockSpec((_BLK, 1), lambda i: (i, 0)),
            pl.BlockSpec((1, h), lambda i: (0, 0)),
            pl.BlockSpec((h, h), lambda i: (0, 0)),
        ],
        out_specs=pl.BlockSpec((_BLK, h), lambda i: (i, 0)),
        out_shape=jax.ShapeDtypeStruct((n, h), F32),
    )(accp, y, dinv, b, w)


def _tc_fin(accp, y, dinv, b, batchf, wfc, bfc, n, h, nout, nblk):
    return pl.pallas_call(
        functools.partial(_tc_fin_body, nblk),
        grid=(nblk,),
        in_specs=[
            pl.BlockSpec((NC, _BLK, h), lambda i: (0, i, 0)),
            pl.BlockSpec((_BLK, h), lambda i: (i, 0)),
            pl.BlockSpec((_BLK, 1), lambda i: (i, 0)),
            pl.BlockSpec((1, h), lambda i: (0, 0)),
            pl.BlockSpec((1, 1, _BLK), lambda i: (i, 0, 0)),
            pl.BlockSpec((h, nout), lambda i: (0, 0)),
            pl.BlockSpec((1, nout), lambda i: (0, 0)),
        ],
        out_specs=pl.BlockSpec((_G, nout), lambda i: (0, 0)),
        out_shape=jax.ShapeDtypeStruct((_G, nout), F32),
        scratch_shapes=[
            pltpu.VMEM((_G, h), F32),
            pltpu.VMEM((_G, h), F32),
        ],
    )(accp, y, dinv, b, batchf, wfc, bfc)


def kernel(x, edge_index, batch, W1, b1, W2, b2, W3, b3, Wfc, bfc):
    n, d = x.shape
    h = W1.shape[1]
    e = edge_index.shape[1]
    nout = Wfc.shape[1]
    assert n % _BLK == 0 and (n // NS) % _DC == 0
    nblk = n // _BLK
    blk = NW * _C
    cpw = ((e + blk - 1) // blk + 3) // 4 * 4
    ep = cpw * blk
    pad_e = ep - e
    assert cpw >= 8
    npad = ((n + NS * _C - 1) // (NS * _C)) * (NS * _C)
    assert npad > n

    src_f = edge_index[0]
    dst_f = edge_index[1]
    if pad_e:
        pad_src = (jnp.arange(pad_e, dtype=jnp.int32) * 977) % n
        src_f = jnp.concatenate([src_f, pad_src])
        dst_f = jnp.concatenate(
            [dst_f, jnp.full((pad_e,), n, dtype=jnp.int32)])
    src3 = src_f.reshape(NW, cpw, _C)
    dst3 = dst_f.reshape(NW, cpw, _C)
    zeros_t = jnp.zeros((_C, _DW), F32)
    ones_t = jnp.ones((_C, _DW), F32)

    deg_call = _make_deg(n, npad, cpw)
    agg_call = _make_agg(n, npad, h, cpw)

    degp = deg_call(dst3, zeros_t, ones_t)
    xw1 = _tc1a(x.astype(F32), W1, n, d, h, nblk)
    y1, dinv = _tc1b(xw1, degp, n, h, nblk)
    p1 = agg_call(y1, src3, dst3)
    y2 = _tc_mid(p1, y1, dinv, b1.reshape(1, h), W2, n, h, nblk)
    p2 = agg_call(y2, src3, dst3)
    y3 = _tc_mid(p2, y2, dinv, b2.reshape(1, h), W3, n, h, nblk)
    p3 = agg_call(y3, src3, dst3)
    batchf = batch.astype(F32).reshape(nblk, 1, _BLK)
    return _tc_fin(p3, y3, dinv, b3.reshape(1, h), batchf, Wfc,
                   bfc.reshape(1, nout), n, h, nout, nblk)


# speedup vs baseline: 1.0866x; 1.0057x over previous
import functools

import jax
import jax.numpy as jnp
from jax import lax
from jax.experimental import pallas as pl
from jax.experimental.pallas import tpu as pltpu
from jax.experimental.pallas import tpu_sc as plsc

NC = 2
NS = 16
NW = NC * NS
_C = 128
_DC = 125
_DW = 16
_BLK = 5000
_G = 64
F32 = jnp.float32


def _deg_body(n, npad, cpw, dst_hbm, zeros_hbm, ones_hbm, degp_hbm,
              didx, zbuf, obuf, deg_sh):
    c = lax.axis_index("c")
    s = lax.axis_index("s")
    wid = c * NS + s
    pltpu.sync_copy(zeros_hbm, zbuf)
    pltpu.sync_copy(ones_hbm, obuf)
    pltpu.sync_copy(dst_hbm.at[wid], didx)
    for k in range(npad // NS // _C):
        pltpu.sync_copy(zbuf, deg_sh.at[pl.ds(s * (npad // NS) + k * _C, _C)])
    plsc.subcore_barrier()

    def step(j, carry):
        pltpu.sync_copy(obuf, deg_sh.at[didx.at[j]], add=True)
        return carry

    lax.fori_loop(0, cpw, step, 0)
    plsc.subcore_barrier()
    for k in range(n // NS // _DC):
        r0 = s * (n // NS) + k * _DC
        pltpu.sync_copy(deg_sh.at[pl.ds(r0, _DC)], zbuf.at[pl.ds(0, _DC)])
        pltpu.sync_copy(zbuf.at[pl.ds(0, _DC)], degp_hbm.at[c, pl.ds(r0, _DC)])


def _agg_body(n, npad, h, cpw, y_hbm, src_hbm, dst_hbm, accp_hbm,
              sidx, didx, rows0, rows1, gsem0, gsem1, ssem, acc_sh):
    c = lax.axis_index("c")
    s = lax.axis_index("s")
    wid = c * NS + s
    half = cpw // 2
    rows = (rows0, rows1)
    gsems = (gsem0, gsem1)

    pltpu.async_copy(src_hbm.at[wid, pl.ds(0, half)], sidx, gsem0)
    pltpu.async_copy(dst_hbm.at[wid, pl.ds(0, half)], didx, gsem1)

    def zrow(i, carry):
        for k8 in range(h // 16):
            rows0[i, pl.ds(k8 * 16, 16)] = jnp.zeros((16,), F32)
        return carry

    lax.fori_loop(0, _C, zrow, 0)
    zch = npad // NS // _C
    for k in range(zch):
        pltpu.async_copy(
            rows0, acc_sh.at[pl.ds(s * (npad // NS) + k * _C, _C)], ssem)
    for k in range(zch):
        pltpu.make_async_copy(rows0, acc_sh.at[pl.ds(0, _C)], ssem).wait()
    pltpu.make_async_copy(src_hbm.at[wid, pl.ds(0, half)], sidx, gsem0).wait()
    pltpu.make_async_copy(dst_hbm.at[wid, pl.ds(0, half)], didx, gsem1).wait()
    plsc.subcore_barrier()

    pltpu.async_copy(y_hbm.at[sidx.at[0]], rows0, gsem0)

    def g_wait(b):
        pltpu.make_async_copy(y_hbm.at[sidx.at[0]], rows[b], gsems[b]).wait()

    def s_wait():
        pltpu.make_async_copy(rows[0], acc_sh.at[didx.at[0]], ssem).wait()

    g_wait(0)
    pltpu.async_copy(rows0, acc_sh.at[didx.at[0]], ssem, add=True)
    pltpu.async_copy(y_hbm.at[sidx.at[1]], rows1, gsem1)

    def pair(i, carry):
        for off in range(2):
            k = 2 * i + 1 + off
            b = (1 + off) % 2
            g_wait(b)
            s_wait()
            if off == 0:
                @pl.when(k == half - 1)
                def _():
                    pltpu.sync_copy(src_hbm.at[wid, pl.ds(half, half)], sidx)
            else:
                @pl.when(k == half)
                def _():
                    pltpu.sync_copy(dst_hbm.at[wid, pl.ds(half, half)], didx)
            kl = k - half * (k // half)
            pltpu.async_copy(rows[b], acc_sh.at[didx.at[kl]], ssem, add=True)
            k1 = k + 1
            kg = k1 - half * (k1 // half)
            pltpu.async_copy(y_hbm.at[sidx.at[kg]], rows[1 - b],
                             gsems[1 - b])
        return carry

    lax.fori_loop(0, (cpw - 2) // 2, pair, 0)
    g_wait((cpw - 1) % 2)
    s_wait()
    pltpu.async_copy(rows[(cpw - 1) % 2],
                     acc_sh.at[didx.at[half - 1]], ssem, add=True)
    s_wait()
    plsc.subcore_barrier()
    nch = n // NS // _DC
    for k in range(nch):
        r0 = s * (n // NS) + k * _DC
        if k >= 2:
            pltpu.make_async_copy(
                rows[k % 2].at[pl.ds(0, _DC)], accp_hbm.at[c, pl.ds(0, _DC)],
                gsems[k % 2]).wait()
        pltpu.sync_copy(acc_sh.at[pl.ds(r0, _DC)], rows[k % 2].at[pl.ds(0, _DC)])
        pltpu.async_copy(rows[k % 2].at[pl.ds(0, _DC)],
                         accp_hbm.at[c, pl.ds(r0, _DC)], gsems[k % 2])
    pltpu.make_async_copy(
        rows[(nch - 2) % 2].at[pl.ds(0, _DC)], accp_hbm.at[c, pl.ds(0, _DC)],
        gsems[(nch - 2) % 2]).wait()
    pltpu.make_async_copy(
        rows[(nch - 1) % 2].at[pl.ds(0, _DC)], accp_hbm.at[c, pl.ds(0, _DC)],
        gsems[(nch - 1) % 2]).wait()


def _make_deg(n, npad, cpw):
    mesh = plsc.VectorSubcoreMesh(core_axis_name="c", subcore_axis_name="s")
    return pl.kernel(
        functools.partial(_deg_body, n, npad, cpw),
        out_type=jax.ShapeDtypeStruct((NC, n, _DW), F32),
        mesh=mesh,
        scratch_types=[
            pltpu.VMEM((cpw, _C), jnp.int32),
            pltpu.VMEM((_C, _DW), F32),
            pltpu.VMEM((_C, _DW), F32),
            pltpu.VMEM_SHARED((npad, _DW), F32),
        ],
        compiler_params=pltpu.CompilerParams(use_tc_tiling_on_sc=False),
    )


def _make_agg(n, npad, h, cpw):
    mesh = plsc.VectorSubcoreMesh(core_axis_name="c", subcore_axis_name="s")
    return pl.kernel(
        functools.partial(_agg_body, n, npad, h, cpw),
        out_type=jax.ShapeDtypeStruct((NC, n, h), F32),
        mesh=mesh,
        scratch_types=[
            pltpu.VMEM((cpw // 2, _C), jnp.int32),
            pltpu.VMEM((cpw // 2, _C), jnp.int32),
            pltpu.VMEM((_C, h), F32),
            pltpu.VMEM((_C, h), F32),
            pltpu.SemaphoreType.DMA,
            pltpu.SemaphoreType.DMA,
            pltpu.SemaphoreType.DMA,
            pltpu.VMEM_SHARED((npad, h), F32),
        ],
        compiler_params=pltpu.CompilerParams(use_tc_tiling_on_sc=False),
    )


def _tc1a_body(x_ref, w_ref, xw_ref):
    xw_ref[...] = jnp.dot(x_ref[...], w_ref[...], preferred_element_type=F32)


def _tc1b_body(xw_ref, degp_ref, y_ref, dinv_ref):
    deg = degp_ref[0, :, 0:1] + degp_ref[1, :, 0:1] + 1.0
    di = lax.rsqrt(deg)
    dinv_ref[...] = di
    y_ref[...] = xw_ref[...] * di


def _tc_mid_body(accp_ref, y_ref, dinv_ref, b_ref, w_ref, out_ref):
    di = dinv_ref[...]
    hcur = jnp.maximum(
        di * (accp_ref[0] + accp_ref[1] + y_ref[...]) + b_ref[...], 0.0)
    out_ref[...] = jnp.dot(hcur, w_ref[...], preferred_element_type=F32) * di


def _tc_fin_body(nblk, accp_ref, y_ref, dinv_ref, b_ref, batch_ref,
                 wfc_ref, bfc_ref, out_ref, sums, counts):
    i = pl.program_id(0)

    @pl.when(i == 0)
    def _():
        sums[...] = jnp.zeros_like(sums)
        counts[...] = jnp.zeros_like(counts)

    di = dinv_ref[...]
    hcur = jnp.maximum(
        di * (accp_ref[0] + accp_ref[1] + y_ref[...]) + b_ref[...], 0.0)
    gid = lax.broadcasted_iota(jnp.int32, (_G, hcur.shape[0]), 0).astype(F32)
    sel_t = (batch_ref[0] == gid).astype(F32)
    sums[...] += lax.dot_general(sel_t, hcur, (((1,), (0,)), ((), ())),
                                 preferred_element_type=F32)
    counts[...] += jnp.broadcast_to(jnp.sum(sel_t, axis=1)[:, None],
                                    counts.shape)

    @pl.when(i == nblk - 1)
    def _():
        pooled = sums[...] / jnp.maximum(counts[...], 1.0)
        out_ref[...] = jnp.dot(pooled, wfc_ref[...],
                               preferred_element_type=F32) + bfc_ref[...]


def _tc1a(x, w, n, d, h, nblk):
    return pl.pallas_call(
        _tc1a_body,
        grid=(nblk,),
        in_specs=[
            pl.BlockSpec((_BLK, d), lambda i: (i, 0)),
            pl.BlockSpec((d, h), lambda i: (0, 0)),
        ],
        out_specs=pl.BlockSpec((_BLK, h), lambda i: (i, 0)),
        out_shape=jax.ShapeDtypeStruct((n, h), F32),
    )(x, w)


def _tc1b(xw, degp, n, h, nblk):
    return pl.pallas_call(
        _tc1b_body,
        grid=(nblk,),
        in_specs=[
            pl.BlockSpec((_BLK, h), lambda i: (i, 0)),
            pl.BlockSpec((NC, _BLK, _DW), lambda i: (0, i, 0)),
        ],
        out_specs=[
            pl.BlockSpec((_BLK, h), lambda i: (i, 0)),
            pl.BlockSpec((_BLK, 1), lambda i: (i, 0)),
        ],
        out_shape=[
            jax.ShapeDtypeStruct((n, h), F32),
            jax.ShapeDtypeStruct((n, 1), F32),
        ],
    )(xw, degp)


def _tc_mid(accp, y, dinv, b, w, n, h, nblk):
    return pl.pallas_call(
        _tc_mid_body,
        grid=(nblk,),
        in_specs=[
            pl.BlockSpec((NC, _BLK, h), lambda i: (0, i, 0)),
            pl.BlockSpec((_BLK, h), lambda i: (i, 0)),
            pl.BlockSpec((_BLK, 1), lambda i: (i, 0)),
            pl.BlockSpec((1, h), lambda i: (0, 0)),
            pl.BlockSpec((h, h), lambda i: (0, 0)),
        ],
        out_specs=pl.BlockSpec((_BLK, h), lambda i: (i, 0)),
        out_shape=jax.ShapeDtypeStruct((n, h), F32),
    )(accp, y, dinv, b, w)


def _tc_fin(accp, y, dinv, b, batchf, wfc, bfc, n, h, nout, nblk):
    return pl.pallas_call(
        functools.partial(_tc_fin_body, nblk),
        grid=(nblk,),
        in_specs=[
            pl.BlockSpec((NC, _BLK, h), lambda i: (0, i, 0)),
            pl.BlockSpec((_BLK, h), lambda i: (i, 0)),
            pl.BlockSpec((_BLK, 1), lambda i: (i, 0)),
            pl.BlockSpec((1, h), lambda i: (0, 0)),
            pl.BlockSpec((1, 1, _BLK), lambda i: (i, 0, 0)),
            pl.BlockSpec((h, nout), lambda i: (0, 0)),
            pl.BlockSpec((1, nout), lambda i: (0, 0)),
        ],
        out_specs=pl.BlockSpec((_G, nout), lambda i: (0, 0)),
        out_shape=jax.ShapeDtypeStruct((_G, nout), F32),
        scratch_shapes=[
            pltpu.VMEM((_G, h), F32),
            pltpu.VMEM((_G, h), F32),
        ],
    )(accp, y, dinv, b, batchf, wfc, bfc)


def kernel(x, edge_index, batch, W1, b1, W2, b2, W3, b3, Wfc, bfc):
    n, d = x.shape
    h = W1.shape[1]
    e = edge_index.shape[1]
    nout = Wfc.shape[1]
    assert n % _BLK == 0 and (n // NS) % _DC == 0
    nblk = n // _BLK
    blk = NW * _C
    cpw = ((e + blk - 1) // blk + 3) // 4 * 4
    ep = cpw * blk
    pad_e = ep - e
    assert cpw >= 8
    npad = ((n + NS * _C - 1) // (NS * _C)) * (NS * _C)
    assert npad > n

    src_f = edge_index[0]
    dst_f = edge_index[1]
    if pad_e:
        pad_src = (jnp.arange(pad_e, dtype=jnp.int32) * 977) % n
        src_f = jnp.concatenate([src_f, pad_src])
        dst_f = jnp.concatenate(
            [dst_f, jnp.full((pad_e,), n, dtype=jnp.int32)])
    src3 = src_f.reshape(NW, cpw, _C)
    dst3 = dst_f.reshape(NW, cpw, _C)
    zeros_t = jnp.zeros((_C, _DW), F32)
    ones_t = jnp.ones((_C, _DW), F32)

    deg_call = _make_deg(n, npad, cpw)
    agg_call = _make_agg(n, npad, h, cpw)

    degp = deg_call(dst3, zeros_t, ones_t)
    xw1 = _tc1a(x.astype(F32), W1, n, d, h, nblk)
    y1, dinv = _tc1b(xw1, degp, n, h, nblk)
    p1 = agg_call(y1, src3, dst3)
    y2 = _tc_mid(p1, y1, dinv, b1.reshape(1, h), W2, n, h, nblk)
    p2 = agg_call(y2, src3, dst3)
    y3 = _tc_mid(p2, y2, dinv, b2.reshape(1, h), W3, n, h, nblk)
    p3 = agg_call(y3, src3, dst3)
    batchf = batch.astype(F32).reshape(nblk, 1, _BLK)
    return _tc_fin(p3, y3, dinv, b3.reshape(1, h), batchf, Wfc,
                   bfc.reshape(1, nout), n, h, nout, nblk)


# speedup vs baseline: 1.0872x; 1.0005x over previous
import functools

import jax
import jax.numpy as jnp
from jax import lax
from jax.experimental import pallas as pl
from jax.experimental.pallas import tpu as pltpu
from jax.experimental.pallas import tpu_sc as plsc

NC = 2
NS = 16
NW = NC * NS
_C = 128
_DC = 125
_DW = 16
_BLK = 5000
_G = 64
F32 = jnp.float32


def _deg_body(n, npad, cpw, dst_hbm, zeros_hbm, ones_hbm, degp_hbm,
              didx, zbuf, obuf, deg_sh):
    c = lax.axis_index("c")
    s = lax.axis_index("s")
    wid = c * NS + s
    pltpu.sync_copy(zeros_hbm, zbuf)
    pltpu.sync_copy(ones_hbm, obuf)
    pltpu.sync_copy(dst_hbm.at[wid], didx)
    for k in range(npad // NS // _C):
        pltpu.sync_copy(zbuf, deg_sh.at[pl.ds(s * (npad // NS) + k * _C, _C)])
    plsc.subcore_barrier()

    def step(j, carry):
        pltpu.sync_copy(obuf, deg_sh.at[didx.at[j]], add=True)
        return carry

    lax.fori_loop(0, cpw, step, 0)
    plsc.subcore_barrier()
    for k in range(n // NS // _DC):
        r0 = s * (n // NS) + k * _DC
        pltpu.sync_copy(deg_sh.at[pl.ds(r0, _DC)], zbuf.at[pl.ds(0, _DC)])
        pltpu.sync_copy(zbuf.at[pl.ds(0, _DC)], degp_hbm.at[c, pl.ds(r0, _DC)])


def _agg_body(n, npad, h, cpw, y_hbm, src_hbm, dst_hbm, accp_hbm,
              sidx, didx, rows0, rows1, gsem0, gsem1, ssem, acc_sh):
    c = lax.axis_index("c")
    s = lax.axis_index("s")
    wid = c * NS + s
    half = cpw // 2
    rows = (rows0, rows1)
    gsems = (gsem0, gsem1)

    pltpu.async_copy(src_hbm.at[wid, pl.ds(0, half)], sidx, gsem0)
    pltpu.async_copy(dst_hbm.at[wid, pl.ds(0, half)], didx, gsem1)

    def zrow(i, carry):
        for k8 in range(h // 16):
            rows0[i, pl.ds(k8 * 16, 16)] = jnp.zeros((16,), F32)
        return carry

    lax.fori_loop(0, _C, zrow, 0)
    zch = npad // NS // _C
    for k in range(zch):
        pltpu.async_copy(
            rows0, acc_sh.at[pl.ds(s * (npad // NS) + k * _C, _C)], ssem)
    for k in range(zch):
        pltpu.make_async_copy(rows0, acc_sh.at[pl.ds(0, _C)], ssem).wait()
    pltpu.make_async_copy(src_hbm.at[wid, pl.ds(0, half)], sidx, gsem0).wait()
    pltpu.make_async_copy(dst_hbm.at[wid, pl.ds(0, half)], didx, gsem1).wait()
    plsc.subcore_barrier()

    pltpu.async_copy(y_hbm.at[sidx.at[0]], rows0, gsem0)

    def g_wait(b):
        pltpu.make_async_copy(y_hbm.at[sidx.at[0]], rows[b], gsems[b]).wait()

    def s_wait():
        pltpu.make_async_copy(rows[0], acc_sh.at[didx.at[0]], ssem).wait()

    g_wait(0)
    pltpu.async_copy(rows0, acc_sh.at[didx.at[0]], ssem, add=True)
    pltpu.async_copy(y_hbm.at[sidx.at[1]], rows1, gsem1)

    def pair(i, carry):
        for off in range(2):
            k = 2 * i + 1 + off
            b = (1 + off) % 2
            g_wait(b)
            s_wait()
            if off == 0:
                @pl.when(k == half - 1)
                def _():
                    pltpu.sync_copy(src_hbm.at[wid, pl.ds(half, half)], sidx)
            else:
                @pl.when(k == half)
                def _():
                    pltpu.sync_copy(dst_hbm.at[wid, pl.ds(half, half)], didx)
            kl = k - half * (k // half)
            pltpu.async_copy(rows[b], acc_sh.at[didx.at[kl]], ssem, add=True)
            k1 = k + 1
            kg = k1 - half * (k1 // half)
            pltpu.async_copy(y_hbm.at[sidx.at[kg]], rows[1 - b],
                             gsems[1 - b])
        return carry

    lax.fori_loop(0, (cpw - 2) // 2, pair, 0)
    g_wait((cpw - 1) % 2)
    s_wait()
    pltpu.async_copy(rows[(cpw - 1) % 2],
                     acc_sh.at[didx.at[half - 1]], ssem, add=True)
    s_wait()
    plsc.subcore_barrier()
    nch = n // NS // _DC
    for k in range(nch):
        r0 = s * (n // NS) + k * _DC
        if k >= 2:
            pltpu.make_async_copy(
                rows[k % 2].at[pl.ds(0, _DC)], accp_hbm.at[c, pl.ds(0, _DC)],
                gsems[k % 2]).wait()
        pltpu.sync_copy(acc_sh.at[pl.ds(r0, _DC)], rows[k % 2].at[pl.ds(0, _DC)])
        pltpu.async_copy(rows[k % 2].at[pl.ds(0, _DC)],
                         accp_hbm.at[c, pl.ds(r0, _DC)], gsems[k % 2])
    pltpu.make_async_copy(
        rows[(nch - 2) % 2].at[pl.ds(0, _DC)], accp_hbm.at[c, pl.ds(0, _DC)],
        gsems[(nch - 2) % 2]).wait()
    pltpu.make_async_copy(
        rows[(nch - 1) % 2].at[pl.ds(0, _DC)], accp_hbm.at[c, pl.ds(0, _DC)],
        gsems[(nch - 1) % 2]).wait()


def _make_deg(n, npad, cpw):
    mesh = plsc.VectorSubcoreMesh(core_axis_name="c", subcore_axis_name="s")
    return pl.kernel(
        functools.partial(_deg_body, n, npad, cpw),
        out_type=jax.ShapeDtypeStruct((NC, n, _DW), F32),
        mesh=mesh,
        scratch_types=[
            pltpu.VMEM((cpw, _C), jnp.int32),
            pltpu.VMEM((_C, _DW), F32),
            pltpu.VMEM((_C, _DW), F32),
            pltpu.VMEM_SHARED((npad, _DW), F32),
        ],
        compiler_params=pltpu.CompilerParams(use_tc_tiling_on_sc=False),
    )


def _make_agg(n, npad, h, cpw):
    mesh = plsc.VectorSubcoreMesh(core_axis_name="c", subcore_axis_name="s")
    return pl.kernel(
        functools.partial(_agg_body, n, npad, h, cpw),
        out_type=jax.ShapeDtypeStruct((NC, n, h), F32),
        mesh=mesh,
        scratch_types=[
            pltpu.VMEM((cpw // 2, _C), jnp.int32),
            pltpu.VMEM((cpw // 2, _C), jnp.int32),
            pltpu.VMEM((_C, h), F32),
            pltpu.VMEM((_C, h), F32),
            pltpu.SemaphoreType.DMA,
            pltpu.SemaphoreType.DMA,
            pltpu.SemaphoreType.DMA,
            pltpu.VMEM_SHARED((npad, h), F32),
        ],
        compiler_params=pltpu.CompilerParams(use_tc_tiling_on_sc=False),
    )


def _tc1a_body(x_ref, w_ref, xw_ref):
    xw_ref[...] = jnp.dot(x_ref[...], w_ref[...], preferred_element_type=F32)


def _tc1b_body(xw_ref, degp_ref, y_ref, dinv_ref):
    deg = degp_ref[0, :, 0:1] + degp_ref[1, :, 0:1] + 1.0
    di = lax.rsqrt(deg)
    dinv_ref[...] = di
    y_ref[...] = xw_ref[...] * di


def _tc_mid_body(accp_ref, y_ref, dinv_ref, b_ref, w_ref, out_ref):
    di = dinv_ref[...]
    hcur = jnp.maximum(
        di * (accp_ref[0] + accp_ref[1] + y_ref[...]) + b_ref[...], 0.0)
    out_ref[...] = jnp.dot(hcur, w_ref[...], preferred_element_type=F32) * di


def _tc_fin_body(nblk, accp_ref, y_ref, dinv_ref, b_ref, batch_ref,
                 wfc_ref, bfc_ref, out_ref, sums, counts):
    i = pl.program_id(0)

    @pl.when(i == 0)
    def _():
        sums[...] = jnp.zeros_like(sums)
        counts[...] = jnp.zeros_like(counts)

    di = dinv_ref[...]
    hcur = jnp.maximum(
        di * (accp_ref[0] + accp_ref[1] + y_ref[...]) + b_ref[...], 0.0)
    gid = lax.broadcasted_iota(jnp.int32, (_G, hcur.shape[0]), 0).astype(F32)
    sel_t = (batch_ref[0] == gid).astype(F32)
    sums[...] += lax.dot_general(sel_t, hcur, (((1,), (0,)), ((), ())),
                                 preferred_element_type=F32)
    counts[...] += jnp.broadcast_to(jnp.sum(sel_t, axis=1)[:, None],
                                    counts.shape)

    @pl.when(i == nblk - 1)
    def _():
        pooled = sums[...] / jnp.maximum(counts[...], 1.0)
        out_ref[...] = jnp.dot(pooled, wfc_ref[...],
                               preferred_element_type=F32) + bfc_ref[...]


def _tc1a(x, w, n, d, h, nblk):
    return pl.pallas_call(
        _tc1a_body,
        grid=(nblk,),
        in_specs=[
            pl.BlockSpec((_BLK, d), lambda i: (i, 0)),
            pl.BlockSpec((d, h), lambda i: (0, 0)),
        ],
        out_specs=pl.BlockSpec((_BLK, h), lambda i: (i, 0)),
        out_shape=jax.ShapeDtypeStruct((n, h), F32),
    )(x, w)


def _tc1b(xw, degp, n, h, nblk):
    return pl.pallas_call(
        _tc1b_body,
        grid=(nblk,),
        in_specs=[
            pl.BlockSpec((_BLK, h), lambda i: (i, 0)),
            pl.BlockSpec((NC, _BLK, _DW), lambda i: (0, i, 0)),
        ],
        out_specs=[
            pl.BlockSpec((_BLK, h), lambda i: (i, 0)),
            pl.BlockSpec((_BLK, 1), lambda i: (i, 0)),
        ],
        out_shape=[
            jax.ShapeDtypeStruct((n, h), F32),
            jax.ShapeDtypeStruct((n, 1), F32),
        ],
    )(xw, degp)


def _tc_mid(accp, y, dinv, b, w, n, h, nblk):
    return pl.pallas_call(
        _tc_mid_body,
        grid=(nblk,),
        in_specs=[
            pl.BlockSpec((NC, _BLK, h), lambda i: (0, i, 0)),
            pl.BlockSpec((_BLK, h), lambda i: (i, 0)),
            pl.BlockSpec((_BLK, 1), lambda i: (i, 0)),
            pl.BlockSpec((1, h), lambda i: (0, 0)),
            pl.BlockSpec((h, h), lambda i: (0, 0)),
        ],
        out_specs=pl.BlockSpec((_BLK, h), lambda i: (i, 0)),
        out_shape=jax.ShapeDtypeStruct((n, h), F32),
    )(accp, y, dinv, b, w)


def _tc_fin(accp, y, dinv, b, batchf, wfc, bfc, n, h, nout, nblk):
    return pl.pallas_call(
        functools.partial(_tc_fin_body, nblk),
        grid=(nblk,),
        in_specs=[
            pl.BlockSpec((NC, _BLK, h), lambda i: (0, i, 0)),
            pl.BlockSpec((_BLK, h), lambda i: (i, 0)),
            pl.BlockSpec((_BLK, 1), lambda i: (i, 0)),
            pl.BlockSpec((1, h), lambda i: (0, 0)),
            pl.BlockSpec((1, 1, _BLK), lambda i: (i, 0, 0)),
            pl.BlockSpec((h, nout), lambda i: (0, 0)),
            pl.BlockSpec((1, nout), lambda i: (0, 0)),
        ],
        out_specs=pl.BlockSpec((_G, nout), lambda i: (0, 0)),
        out_shape=jax.ShapeDtypeStruct((_G, nout), F32),
        scratch_shapes=[
            pltpu.VMEM((_G, h), F32),
            pltpu.VMEM((_G, h), F32),
        ],
    )(accp, y, dinv, b, batchf, wfc, bfc)


def kernel(x, edge_index, batch, W1, b1, W2, b2, W3, b3, Wfc, bfc):
    n, d = x.shape
    h = W1.shape[1]
    e = edge_index.shape[1]
    nout = Wfc.shape[1]
    assert n % _BLK == 0 and (n // NS) % _DC == 0
    nblk = n // _BLK
    blk = NW * _C
    cpw = ((e + blk - 1) // blk + 3) // 4 * 4
    ep = cpw * blk
    pad_e = ep - e
    assert cpw >= 8
    npad = ((n + NS * _C - 1) // (NS * _C)) * (NS * _C)
    assert npad > n

    src_f = edge_index[0]
    dst_f = edge_index[1]
    if pad_e:
        pad_src = (jnp.arange(pad_e, dtype=jnp.int32) * 977) % n
        src_f = jnp.concatenate([src_f, pad_src])
        dst_f = jnp.concatenate(
            [dst_f, jnp.full((pad_e,), n, dtype=jnp.int32)])
    dst3 = dst_f.reshape(NW, cpw, _C)
    src3 = lax.optimization_barrier(src_f.reshape(NW, cpw, _C))
    zeros_t = jnp.zeros((_C, _DW), F32)
    ones_t = jnp.ones((_C, _DW), F32)

    deg_call = _make_deg(n, npad, cpw)
    agg_call = _make_agg(n, npad, h, cpw)

    degp = deg_call(dst3, zeros_t, ones_t)
    xw1 = _tc1a(x.astype(F32), W1, n, d, h, nblk)
    y1, dinv = _tc1b(xw1, degp, n, h, nblk)
    p1 = agg_call(y1, src3, dst3)
    y2 = _tc_mid(p1, y1, dinv, b1.reshape(1, h), W2, n, h, nblk)
    p2 = agg_call(y2, src3, dst3)
    y3 = _tc_mid(p2, y2, dinv, b2.reshape(1, h), W3, n, h, nblk)
    p3 = agg_call(y3, src3, dst3)
    batchf = batch.astype(F32).reshape(nblk, 1, _BLK)
    return _tc_fin(p3, y3, dinv, b3.reshape(1, h), batchf, Wfc,
                   bfc.reshape(1, nout), n, h, nout, nblk)
